# baseline (speedup 1.0000x reference)
import jax
import jax.numpy as jnp
import numpy as np
from jax import lax
from jax.experimental import pallas as pl
from jax.experimental.pallas import tpu as pltpu
from jax.experimental.pallas import tpu_sc as plsc

N_ATOMS = 2048
N_BONDS = 2048
N_ANGLES = 4096

NS = 16
BP = N_BONDS // NS
AP = N_ANGLES // NS
BG = BP // 16
AG = AP // 16

NROWS = (N_ATOMS * N_ATOMS) // 128

ACC = 8192
CHUNK = ACC // NS
E_SLOT = N_ATOMS * 3
E_TILE = E_SLOT // CHUNK

FMAX = np.float32(3.4028235e38)
PI = np.float32(3.14159265358979)
ACOS_C = [1.5707963050, -0.2145988016, 0.0889789874, -0.0501743046,
          0.0308918810, -0.0170881256, 0.0066700901, -0.0012624911]


def _sqrt(y):
    i = plsc.bitcast(y, jnp.int32)
    i = jnp.int32(0x5F3759DF) - (i >> 1)
    r = plsc.bitcast(i, jnp.float32)
    for _ in range(3):
        r = r * (1.5 - 0.5 * y * r * r)
    return y * r


def _acos(c):
    xa = jnp.abs(c)
    p = jnp.float32(ACOS_C[7])
    for a in ACOS_C[6::-1]:
        p = p * xa + jnp.float32(a)
    t = _sqrt(1.0 - xa) * p
    return jnp.where(c < 0, PI - t, t)


def _trow(a, b):
    return ((a >> 3) << 7) + ((b >> 7) << 3) + (a & 7)


def _sc_body(dist4_hbm, vec5_hbm, bidx_hbm, bpar_hbm, aidx_hbm, apar_hbm,
             out_hbm,
             ba_v, bb_v, bk0_v, breq_v,
             a1_v, a2_v, a3_v, ak0_v, ath0_v,
             bidx_v, aidx_v, blane_v, lane1_v, lane3_v,
             rb0_v, rb1_v,
             dvalb_v, dval21_v, dval23_v,
             vbx_v, vby_v, vbz_v,
             v21x_v, v21y_v, v21z_v,
             v23x_v, v23y_v, v23z_v,
             acc_v, blk_v, spmem, sem, sem2):
    wid = lax.axis_index("s")
    dist_t = dist4_hbm.reshape(NROWS, 128)
    vec_t = vec5_hbm.reshape(3 * NROWS, 128)

    pltpu.sync_copy(bidx_hbm.at[pl.ds(wid * 256, 128)], ba_v)
    pltpu.sync_copy(bidx_hbm.at[pl.ds(wid * 256 + 128, 128)], bb_v)
    pltpu.sync_copy(bpar_hbm.at[pl.ds(wid * 256, 128)], bk0_v)
    pltpu.sync_copy(bpar_hbm.at[pl.ds(wid * 256 + 128, 128)], breq_v)
    for h in range(2):
        g = 2 * wid + h
        dst = pl.ds(h * 128, 128)
        pltpu.sync_copy(aidx_hbm.at[pl.ds(g * 512, 128)], a1_v.at[dst])
        pltpu.sync_copy(aidx_hbm.at[pl.ds(g * 512 + 128, 128)], a2_v.at[dst])
        pltpu.sync_copy(aidx_hbm.at[pl.ds(g * 512 + 256, 128)], a3_v.at[dst])
        pltpu.sync_copy(apar_hbm.at[pl.ds(g * 256, 128)], ak0_v.at[dst])
        pltpu.sync_copy(apar_hbm.at[pl.ds(g * 256 + 128, 128)], ath0_v.at[dst])

    for j in range(BG):
        s = pl.ds(j * 16, 16)
        a = ba_v[s]
        b = bb_v[s]
        t = _trow(a, b)
        bidx_v[0, s] = t
        bidx_v[1, s] = t
        bidx_v[2, s] = t + NROWS
        bidx_v[3, s] = t + 2 * NROWS
        blane_v[s] = b & 127
    for j in range(AG):
        s = pl.ds(j * 16, 16)
        i1 = a1_v[s]
        i2 = a2_v[s]
        i3 = a3_v[s]
        t21 = _trow(i2, i1)
        t23 = _trow(i2, i3)
        aidx_v[0, s] = t21
        aidx_v[1, s] = t23
        aidx_v[2, s] = t21
        aidx_v[3, s] = t23
        aidx_v[4, s] = t21 + NROWS
        aidx_v[5, s] = t23 + NROWS
        aidx_v[6, s] = t21 + 2 * NROWS
        aidx_v[7, s] = t23 + 2 * NROWS
        lane1_v[s] = i1 & 127
        lane3_v[s] = i3 & 127

    def _zero(i, _):
        acc_v[pl.ds(i * 16, 16)] = jnp.zeros((16,), jnp.float32)
        return _
    lax.fori_loop(0, ACC // 16, _zero, None)

    passes = [
        (dist_t, True, 0, dvalb_v, blane_v),
        (vec_t, True, 1, vbx_v, blane_v),
        (vec_t, True, 2, vby_v, blane_v),
        (vec_t, True, 3, vbz_v, blane_v),
        (dist_t, False, 0, dval21_v, lane1_v),
        (dist_t, False, 1, dval23_v, lane3_v),
        (vec_t, False, 2, v21x_v, lane1_v),
        (vec_t, False, 3, v23x_v, lane3_v),
        (vec_t, False, 4, v21y_v, lane1_v),
        (vec_t, False, 5, v23y_v, lane3_v),
        (vec_t, False, 6, v21z_v, lane1_v),
        (vec_t, False, 7, v23z_v, lane3_v),
    ]

    def _fire(i):
        tbl, is_bond, k, _, _ = passes[i]
        buf = rb0_v if i % 2 == 0 else rb1_v
        s = sem if i % 2 == 0 else sem2
        if is_bond:
            return [pltpu.async_copy(
                tbl.at[bidx_v.at[k]], buf.at[pl.ds(0, 128)], s)]
        return [
            pltpu.async_copy(tbl.at[aidx_v.at[k, pl.ds(0, 128)]],
                             buf.at[pl.ds(0, 128)], s),
            pltpu.async_copy(tbl.at[aidx_v.at[k, pl.ds(128, 128)]],
                             buf.at[pl.ds(128, 128)], s),
        ]

    iota = lax.iota(jnp.int32, 16)
    pend = _fire(0)
    for i in range(len(passes)):
        nxt = _fire(i + 1) if i + 1 < len(passes) else []
        for c in pend:
            c.wait()
        _, is_bond, _, dst, laner = passes[i]
        buf = rb0_v if i % 2 == 0 else rb1_v
        for g in range(BG if is_bond else AG):
            s = pl.ds(g * 16, 16)
            dst[s] = plsc.load_gather(buf, [iota + g * 16, laner[s]])
        pend = nxt

    evec = jnp.zeros((16,), jnp.float32)

    for j in range(BG):
        s = pl.ds(j * 16, 16)
        d = dvalb_v[s]
        k0 = bk0_v[s]
        x = d - breq_v[s]
        evec = evec + k0 * x * x
        f = 2.0 * k0 * x
        fx = f * vbx_v[s]
        fy = f * vby_v[s]
        fz = f * vbz_v[s]
        ia = ba_v[s] * 3
        ib = bb_v[s] * 3
        plsc.addupdate_scatter(acc_v, [ia], fx)
        plsc.addupdate_scatter(acc_v, [ia + 1], fy)
        plsc.addupdate_scatter(acc_v, [ia + 2], fz)
        plsc.addupdate_scatter(acc_v, [ib], -fx)
        plsc.addupdate_scatter(acc_v, [ib + 1], -fy)
        plsc.addupdate_scatter(acc_v, [ib + 2], -fz)

    for j in range(AG):
        s = pl.ds(j * 16, 16)
        d21 = dval21_v[s]
        d23 = dval23_v[s]
        x21 = v21x_v[s]
        y21 = v21y_v[s]
        z21 = v21z_v[s]
        x23 = v23x_v[s]
        y23 = v23y_v[s]
        z23 = v23z_v[s]
        cos = x21 * x23 + y21 * y23 + z21 * z23
        cos = jnp.minimum(jnp.maximum(cos, -1.0), 1.0)
        theta = _acos(cos)
        k0 = ak0_v[s]
        dth = theta - ath0_v[s]
        evec = evec + k0 * dth * dth
        sin = _sqrt(1.0 - cos * cos)
        coef = (-2.0 * k0 * dth) / sin
        coef = jnp.where(coef != coef, jnp.float32(0.0), coef)
        coef = jnp.minimum(jnp.maximum(coef, -FMAX), FMAX)
        c21 = coef / d21
        c23 = coef / d23
        f0x = c21 * (cos * x21 - x23)
        f0y = c21 * (cos * y21 - y23)
        f0z = c21 * (cos * z21 - z23)
        f2x = c23 * (cos * x23 - x21)
        f2y = c23 * (cos * y23 - y21)
        f2z = c23 * (cos * z23 - z21)
        i1 = a1_v[s] * 3
        i2 = a2_v[s] * 3
        i3 = a3_v[s] * 3
        plsc.addupdate_scatter(acc_v, [i1], f0x)
        plsc.addupdate_scatter(acc_v, [i1 + 1], f0y)
        plsc.addupdate_scatter(acc_v, [i1 + 2], f0z)
        plsc.addupdate_scatter(acc_v, [i2], -(f0x + f2x))
        plsc.addupdate_scatter(acc_v, [i2 + 1], -(f0y + f2y))
        plsc.addupdate_scatter(acc_v, [i2 + 2], -(f0z + f2z))
        plsc.addupdate_scatter(acc_v, [i3], f2x)
        plsc.addupdate_scatter(acc_v, [i3 + 1], f2y)
        plsc.addupdate_scatter(acc_v, [i3 + 2], f2z)

    acc_v[pl.ds(E_SLOT, 16)] = evec

    for c in range(NS):
        pltpu.sync_copy(acc_v.at[pl.ds(c * CHUNK, CHUNK)],
                        spmem.at[c, wid])
    plsc.subcore_barrier()
    pltpu.sync_copy(spmem.at[wid], blk_v)

    def _sum(i, _):
        off = i * 16
        tot = blk_v[0, pl.ds(off, 16)]
        for t in range(1, NS):
            tot = tot + blk_v[t, pl.ds(off, 16)]
        blk_v[0, pl.ds(off, 16)] = tot
        return _
    lax.fori_loop(0, CHUNK // 16, _sum, None)

    @pl.when(wid == E_TILE)
    def _finish_energy():
        loc = pl.ds(E_SLOT - E_TILE * CHUNK, 16)
        ev = blk_v[0, loc]
        blk_v[0, loc] = jnp.broadcast_to(jnp.sum(ev), (16,))

    pltpu.sync_copy(blk_v.at[0], out_hbm.at[pl.ds(wid * CHUNK, CHUNK)])


@jax.jit
def kernel(dist_mat, vector_mat, bond_params, angle_params, bond_idx, angle_idx):
    mesh = plsc.VectorSubcoreMesh(
        core_axis_name="c", subcore_axis_name="s", num_cores=1)
    sc_fn = pl.kernel(
        _sc_body,
        out_type=jax.ShapeDtypeStruct((ACC,), jnp.float32),
        mesh=mesh,
        compiler_params=pltpu.CompilerParams(needs_layout_passes=False),
        scratch_types=[
            pltpu.VMEM((BP,), jnp.int32),
            pltpu.VMEM((BP,), jnp.int32),
            pltpu.VMEM((BP,), jnp.float32),
            pltpu.VMEM((BP,), jnp.float32),
            pltpu.VMEM((AP,), jnp.int32),
            pltpu.VMEM((AP,), jnp.int32),
            pltpu.VMEM((AP,), jnp.int32),
            pltpu.VMEM((AP,), jnp.float32),
            pltpu.VMEM((AP,), jnp.float32),
            pltpu.VMEM((4, BP), jnp.int32),
            pltpu.VMEM((8, AP), jnp.int32),
            pltpu.VMEM((BP,), jnp.int32),
            pltpu.VMEM((AP,), jnp.int32),
            pltpu.VMEM((AP,), jnp.int32),
            pltpu.VMEM((AP, 128), jnp.float32),
            pltpu.VMEM((AP, 128), jnp.float32),
            pltpu.VMEM((BP,), jnp.float32),
            pltpu.VMEM((AP,), jnp.float32),
            pltpu.VMEM((AP,), jnp.float32),
            pltpu.VMEM((BP,), jnp.float32),
            pltpu.VMEM((BP,), jnp.float32),
            pltpu.VMEM((BP,), jnp.float32),
            pltpu.VMEM((AP,), jnp.float32),
            pltpu.VMEM((AP,), jnp.float32),
            pltpu.VMEM((AP,), jnp.float32),
            pltpu.VMEM((AP,), jnp.float32),
            pltpu.VMEM((AP,), jnp.float32),
            pltpu.VMEM((AP,), jnp.float32),
            pltpu.VMEM((ACC,), jnp.float32),
            pltpu.VMEM((NS, CHUNK), jnp.float32),
            pltpu.VMEM_SHARED((NS, NS, CHUNK), jnp.float32),
            pltpu.SemaphoreType.DMA,
            pltpu.SemaphoreType.DMA,
        ],
    )
    f32 = jnp.float32
    i32 = jnp.int32
    dist4 = dist_mat.reshape(N_ATOMS // 8, 8, N_ATOMS // 128, 128
                             ).transpose(0, 2, 1, 3)
    vec5 = vector_mat.transpose(2, 0, 1).reshape(
        3, N_ATOMS // 8, 8, N_ATOMS // 128, 128).transpose(0, 1, 3, 2, 4)
    bidx_f = bond_idx.astype(i32).reshape(16, 128, 2
                                          ).transpose(0, 2, 1).reshape(-1)
    bpar_f = bond_params.astype(f32).reshape(16, 128, 2
                                             ).transpose(0, 2, 1).reshape(-1)
    aidx_p = jnp.pad(angle_idx.astype(i32), ((0, 0), (0, 1)))
    aidx_f = aidx_p.reshape(32, 128, 4).transpose(0, 2, 1).reshape(-1)
    apar_f = angle_params.astype(f32).reshape(32, 128, 2
                                              ).transpose(0, 2, 1).reshape(-1)
    out = sc_fn(dist4, vec5, bidx_f, bpar_f, aidx_f, apar_f)
    energy = out[E_SLOT]
    forces = out[:N_ATOMS * 3].reshape(N_ATOMS, 3)
    return energy, forces


# speedup vs baseline: 1.0092x; 1.0092x over previous
import jax
import jax.numpy as jnp
import numpy as np
from jax import lax
from jax.experimental import pallas as pl
from jax.experimental.pallas import tpu as pltpu
from jax.experimental.pallas import tpu_sc as plsc

N_ATOMS = 2048
N_BONDS = 2048
N_ANGLES = 4096

NS = 16
BP = N_BONDS // NS
AP = N_ANGLES // NS
BG = BP // 16
AG = AP // 16

NROWS = (N_ATOMS * N_ATOMS) // 128

FSLOTS = (N_ATOMS // 128) * 4 * 128
ACC = 10240
CHUNK = ACC // NS
E_SLOT = FSLOTS
E_TILE = E_SLOT // CHUNK
NBUF = 4

FMAX = np.float32(3.4028235e38)
PI = np.float32(3.14159265358979)
ACOS_C = [1.5707963050, -0.2145988016, 0.0889789874, -0.0501743046,
          0.0308918810, -0.0170881256, 0.0066700901, -0.0012624911]


def _sqrt(y):
    i = plsc.bitcast(y, jnp.int32)
    i = jnp.int32(0x5F3759DF) - (i >> 1)
    r = plsc.bitcast(i, jnp.float32)
    for _ in range(3):
        r = r * (1.5 - 0.5 * y * r * r)
    return y * r


def _acos(c):
    xa = jnp.abs(c)
    p = jnp.float32(ACOS_C[7])
    for a in ACOS_C[6::-1]:
        p = p * xa + jnp.float32(a)
    t = _sqrt(1.0 - xa) * p
    return jnp.where(c < 0, PI - t, t)


def _trow(a, b):
    return ((a >> 3) << 7) + ((b >> 7) << 3) + (a & 7)


def _sc_body(dist4_hbm, vec5_hbm, bidx_hbm, bpar_hbm, aidx_hbm, apar_hbm,
             out_hbm,
             ba_v, bb_v, bk0_v, breq_v,
             a1_v, a2_v, a3_v, ak0_v, ath0_v,
             bidx_v, aidx_v, blane_v, lane1_v, lane3_v,
             rb0_v, rb1_v, rb2_v, rb3_v,
             dvalb_v, dval21_v, dval23_v,
             vbx_v, vby_v, vbz_v,
             v21x_v, v21y_v, v21z_v,
             v23x_v, v23y_v, v23z_v,
             acc_v, blk_v, spmem, sem0, sem1, sem2, sem3):
    wid = lax.axis_index("s")
    dist_t = dist4_hbm.reshape(NROWS, 128)
    vec_t = vec5_hbm.reshape(3 * NROWS, 128)

    pltpu.sync_copy(bidx_hbm.at[pl.ds(wid * 256, 128)], ba_v)
    pltpu.sync_copy(bidx_hbm.at[pl.ds(wid * 256 + 128, 128)], bb_v)
    pltpu.sync_copy(bpar_hbm.at[pl.ds(wid * 256, 128)], bk0_v)
    pltpu.sync_copy(bpar_hbm.at[pl.ds(wid * 256 + 128, 128)], breq_v)
    for h in range(2):
        g = 2 * wid + h
        dst = pl.ds(h * 128, 128)
        pltpu.sync_copy(aidx_hbm.at[pl.ds(g * 512, 128)], a1_v.at[dst])
        pltpu.sync_copy(aidx_hbm.at[pl.ds(g * 512 + 128, 128)], a2_v.at[dst])
        pltpu.sync_copy(aidx_hbm.at[pl.ds(g * 512 + 256, 128)], a3_v.at[dst])
        pltpu.sync_copy(apar_hbm.at[pl.ds(g * 256, 128)], ak0_v.at[dst])
        pltpu.sync_copy(apar_hbm.at[pl.ds(g * 256 + 128, 128)], ath0_v.at[dst])

    for j in range(BG):
        s = pl.ds(j * 16, 16)
        a = ba_v[s]
        b = bb_v[s]
        t = _trow(a, b)
        bidx_v[0, s] = t
        bidx_v[1, s] = t
        bidx_v[2, s] = t + NROWS
        bidx_v[3, s] = t + 2 * NROWS
        blane_v[s] = b & 127
    for j in range(AG):
        s = pl.ds(j * 16, 16)
        i1 = a1_v[s]
        i2 = a2_v[s]
        i3 = a3_v[s]
        t21 = _trow(i2, i1)
        t23 = _trow(i2, i3)
        aidx_v[0, s] = t21
        aidx_v[1, s] = t23
        aidx_v[2, s] = t21
        aidx_v[3, s] = t23
        aidx_v[4, s] = t21 + NROWS
        aidx_v[5, s] = t23 + NROWS
        aidx_v[6, s] = t21 + 2 * NROWS
        aidx_v[7, s] = t23 + 2 * NROWS
        lane1_v[s] = i1 & 127
        lane3_v[s] = i3 & 127

    def _zero(i, _):
        acc_v[pl.ds(i * 16, 16)] = jnp.zeros((16,), jnp.float32)
        return _
    lax.fori_loop(0, ACC // 16, _zero, None)

    rbufs = [rb0_v, rb1_v, rb2_v, rb3_v]
    sems = [sem0, sem1, sem2, sem3]
    passes = [
        (dist_t, True, 0, 0, dvalb_v, blane_v),
        (vec_t, True, 1, 0, vbx_v, blane_v),
        (vec_t, True, 2, 0, vby_v, blane_v),
        (vec_t, True, 3, 0, vbz_v, blane_v),
    ]
    for k, dst, laner in ((0, dval21_v, lane1_v), (1, dval23_v, lane3_v),
                          (2, v21x_v, lane1_v), (3, v23x_v, lane3_v),
                          (4, v21y_v, lane1_v), (5, v23y_v, lane3_v),
                          (6, v21z_v, lane1_v), (7, v23z_v, lane3_v)):
        for h in range(2):
            passes.append((dist_t if k < 2 else vec_t, False, k, h,
                           dst, laner))
    NP = len(passes)

    def _fire(i):
        tbl, is_bond, k, h, _, _ = passes[i]
        buf = rbufs[i % NBUF]
        s = sems[i % NBUF]
        if is_bond:
            idx = bidx_v.at[k]
        else:
            idx = aidx_v.at[k, pl.ds(h * 128, 128)]
        return pltpu.async_copy(tbl.at[idx], buf, s)

    iota = lax.iota(jnp.int32, 16)
    cps = {i: _fire(i) for i in range(min(NBUF - 1, NP))}
    for i in range(NP):
        cps[i].wait()
        _, _, _, h, dst, laner = passes[i]
        buf = rbufs[i % NBUF]
        base = h * 128
        for g in range(BG):
            so = pl.ds(base + g * 16, 16)
            dst[so] = plsc.load_gather(buf, [iota + g * 16, laner[so]])
        if i + NBUF - 1 < NP:
            cps[i + NBUF - 1] = _fire(i + NBUF - 1)

    evec = jnp.zeros((16,), jnp.float32)

    for j in range(BG):
        s = pl.ds(j * 16, 16)
        d = dvalb_v[s]
        k0 = bk0_v[s]
        x = d - breq_v[s]
        evec = evec + k0 * x * x
        f = 2.0 * k0 * x
        fx = f * vbx_v[s]
        fy = f * vby_v[s]
        fz = f * vbz_v[s]
        a = ba_v[s]
        b = bb_v[s]
        ia = ((a >> 7) << 9) + (a & 127)
        ib = ((b >> 7) << 9) + (b & 127)
        plsc.addupdate_scatter(acc_v, [ia], fx)
        plsc.addupdate_scatter(acc_v, [ia + 128], fy)
        plsc.addupdate_scatter(acc_v, [ia + 256], fz)
        plsc.addupdate_scatter(acc_v, [ib], -fx)
        plsc.addupdate_scatter(acc_v, [ib + 128], -fy)
        plsc.addupdate_scatter(acc_v, [ib + 256], -fz)

    for j in range(AG):
        s = pl.ds(j * 16, 16)
        d21 = dval21_v[s]
        d23 = dval23_v[s]
        x21 = v21x_v[s]
        y21 = v21y_v[s]
        z21 = v21z_v[s]
        x23 = v23x_v[s]
        y23 = v23y_v[s]
        z23 = v23z_v[s]
        cos = x21 * x23 + y21 * y23 + z21 * z23
        cos = jnp.minimum(jnp.maximum(cos, -1.0), 1.0)
        theta = _acos(cos)
        k0 = ak0_v[s]
        dth = theta - ath0_v[s]
        evec = evec + k0 * dth * dth
        sin = _sqrt(1.0 - cos * cos)
        coef = (-2.0 * k0 * dth) / sin
        coef = jnp.where(coef != coef, jnp.float32(0.0), coef)
        coef = jnp.minimum(jnp.maximum(coef, -FMAX), FMAX)
        c21 = coef / d21
        c23 = coef / d23
        f0x = c21 * (cos * x21 - x23)
        f0y = c21 * (cos * y21 - y23)
        f0z = c21 * (cos * z21 - z23)
        f2x = c23 * (cos * x23 - x21)
        f2y = c23 * (cos * y23 - y21)
        f2z = c23 * (cos * z23 - z21)
        w1 = a1_v[s]
        w2 = a2_v[s]
        w3 = a3_v[s]
        i1 = ((w1 >> 7) << 9) + (w1 & 127)
        i2 = ((w2 >> 7) << 9) + (w2 & 127)
        i3 = ((w3 >> 7) << 9) + (w3 & 127)
        plsc.addupdate_scatter(acc_v, [i1], f0x)
        plsc.addupdate_scatter(acc_v, [i1 + 128], f0y)
        plsc.addupdate_scatter(acc_v, [i1 + 256], f0z)
        plsc.addupdate_scatter(acc_v, [i2], -(f0x + f2x))
        plsc.addupdate_scatter(acc_v, [i2 + 128], -(f0y + f2y))
        plsc.addupdate_scatter(acc_v, [i2 + 256], -(f0z + f2z))
        plsc.addupdate_scatter(acc_v, [i3], f2x)
        plsc.addupdate_scatter(acc_v, [i3 + 128], f2y)
        plsc.addupdate_scatter(acc_v, [i3 + 256], f2z)

    acc_v[pl.ds(E_SLOT, 16)] = evec

    for c in range(NS):
        pltpu.sync_copy(acc_v.at[pl.ds(c * CHUNK, CHUNK)],
                        spmem.at[c, wid])
    plsc.subcore_barrier()
    pltpu.sync_copy(spmem.at[wid], blk_v)

    def _sum(i, _):
        off = i * 16
        tot = blk_v[0, pl.ds(off, 16)]
        for t in range(1, NS):
            tot = tot + blk_v[t, pl.ds(off, 16)]
        blk_v[0, pl.ds(off, 16)] = tot
        return _
    lax.fori_loop(0, CHUNK // 16, _sum, None)

    @pl.when(wid == E_TILE)
    def _finish_energy():
        loc = pl.ds(E_SLOT - E_TILE * CHUNK, 16)
        ev = blk_v[0, loc]
        blk_v[0, loc] = jnp.broadcast_to(jnp.sum(ev), (16,))

    pltpu.sync_copy(blk_v.at[0], out_hbm.at[pl.ds(wid * CHUNK, CHUNK)])


@jax.jit
def kernel(dist_mat, vector_mat, bond_params, angle_params, bond_idx, angle_idx):
    mesh = plsc.VectorSubcoreMesh(
        core_axis_name="c", subcore_axis_name="s", num_cores=1)
    sc_fn = pl.kernel(
        _sc_body,
        out_type=jax.ShapeDtypeStruct((ACC,), jnp.float32),
        mesh=mesh,
        compiler_params=pltpu.CompilerParams(needs_layout_passes=False),
        scratch_types=[
            pltpu.VMEM((BP,), jnp.int32),
            pltpu.VMEM((BP,), jnp.int32),
            pltpu.VMEM((BP,), jnp.float32),
            pltpu.VMEM((BP,), jnp.float32),
            pltpu.VMEM((AP,), jnp.int32),
            pltpu.VMEM((AP,), jnp.int32),
            pltpu.VMEM((AP,), jnp.int32),
            pltpu.VMEM((AP,), jnp.float32),
            pltpu.VMEM((AP,), jnp.float32),
            pltpu.VMEM((4, BP), jnp.int32),
            pltpu.VMEM((8, AP), jnp.int32),
            pltpu.VMEM((BP,), jnp.int32),
            pltpu.VMEM((AP,), jnp.int32),
            pltpu.VMEM((AP,), jnp.int32),
            pltpu.VMEM((128, 128), jnp.float32),
            pltpu.VMEM((128, 128), jnp.float32),
            pltpu.VMEM((128, 128), jnp.float32),
            pltpu.VMEM((128, 128), jnp.float32),
            pltpu.VMEM((BP,), jnp.float32),
            pltpu.VMEM((AP,), jnp.float32),
            pltpu.VMEM((AP,), jnp.float32),
            pltpu.VMEM((BP,), jnp.float32),
            pltpu.VMEM((BP,), jnp.float32),
            pltpu.VMEM((BP,), jnp.float32),
            pltpu.VMEM((AP,), jnp.float32),
            pltpu.VMEM((AP,), jnp.float32),
            pltpu.VMEM((AP,), jnp.float32),
            pltpu.VMEM((AP,), jnp.float32),
            pltpu.VMEM((AP,), jnp.float32),
            pltpu.VMEM((AP,), jnp.float32),
            pltpu.VMEM((ACC,), jnp.float32),
            pltpu.VMEM((NS, CHUNK), jnp.float32),
            pltpu.VMEM_SHARED((NS, NS, CHUNK), jnp.float32),
            pltpu.SemaphoreType.DMA,
            pltpu.SemaphoreType.DMA,
            pltpu.SemaphoreType.DMA,
            pltpu.SemaphoreType.DMA,
        ],
    )
    f32 = jnp.float32
    i32 = jnp.int32
    dist4 = dist_mat.reshape(N_ATOMS // 8, 8, N_ATOMS // 128, 128
                             ).transpose(0, 2, 1, 3)
    vec5 = vector_mat.transpose(2, 0, 1).reshape(
        3, N_ATOMS // 8, 8, N_ATOMS // 128, 128).transpose(0, 1, 3, 2, 4)
    bidx_f = bond_idx.astype(i32).reshape(16, 128, 2
                                          ).transpose(0, 2, 1).reshape(-1)
    bpar_f = bond_params.astype(f32).reshape(16, 128, 2
                                             ).transpose(0, 2, 1).reshape(-1)
    aidx_p = jnp.pad(angle_idx.astype(i32), ((0, 0), (0, 1)))
    aidx_f = aidx_p.reshape(32, 128, 4).transpose(0, 2, 1).reshape(-1)
    apar_f = angle_params.astype(f32).reshape(32, 128, 2
                                              ).transpose(0, 2, 1).reshape(-1)
    out = sc_fn(dist4, vec5, bidx_f, bpar_f, aidx_f, apar_f)
    energy = out[E_SLOT]
    forces = out[:FSLOTS].reshape(16, 4, 128).transpose(0, 2, 1
                                                        ).reshape(N_ATOMS, 4)[:, :3]
    return energy, forces


# speedup vs baseline: 1.3084x; 1.2964x over previous
import jax
import jax.numpy as jnp
import numpy as np
from jax import lax
from jax.experimental import pallas as pl
from jax.experimental.pallas import tpu as pltpu
from jax.experimental.pallas import tpu_sc as plsc

N_ATOMS = 2048
N_BONDS = 2048
N_ANGLES = 4096

NC = 2
NS = 16
NW = NC * NS
BP = N_BONDS // NW
AP = N_ANGLES // NW
BG = BP // 16
AG = AP // 16

NROWS = (N_ATOMS * N_ATOMS) // 128

FSLOTS = (N_ATOMS // 128) * 4 * 128
ACC = 10240
CHUNK = ACC // NS
E_SLOT = FSLOTS
E_TILE = E_SLOT // CHUNK
NBUF = 4

FMAX = np.float32(3.4028235e38)
PI = np.float32(3.14159265358979)
ACOS_C = [1.5707963050, -0.2145988016, 0.0889789874, -0.0501743046,
          0.0308918810, -0.0170881256, 0.0066700901, -0.0012624911]


def _sqrt(y):
    i = plsc.bitcast(y, jnp.int32)
    i = jnp.int32(0x5F3759DF) - (i >> 1)
    r = plsc.bitcast(i, jnp.float32)
    for _ in range(3):
        r = r * (1.5 - 0.5 * y * r * r)
    return y * r


def _acos(c):
    xa = jnp.abs(c)
    p = jnp.float32(ACOS_C[7])
    for a in ACOS_C[6::-1]:
        p = p * xa + jnp.float32(a)
    t = _sqrt(1.0 - xa) * p
    return jnp.where(c < 0, PI - t, t)


def _trow(a, b):
    return ((a >> 3) << 7) + ((b >> 7) << 3) + (a & 7)


def _sc_body(dist4_hbm, vec5_hbm, bidx_hbm, bpar_hbm, aidx_hbm, apar_hbm,
             out_hbm,
             ba_v, bb_v, bk0_v, breq_v,
             a1_v, a2_v, a3_v, ak0_v, ath0_v,
             bidx_v, aidx_v, blane_v, lane1_v, lane3_v,
             rb0_v, rb1_v, rb2_v, rb3_v,
             dvalb_v, dval21_v, dval23_v,
             vbx_v, vby_v, vbz_v,
             v21x_v, v21y_v, v21z_v,
             v23x_v, v23y_v, v23z_v,
             acc_v, blk_v, spmem, sem0, sem1, sem2, sem3):
    cid = lax.axis_index("c")
    sid = lax.axis_index("s")
    eid = cid * NS + sid
    dist_t = dist4_hbm.reshape(NROWS, 128)
    vec_t = vec5_hbm.reshape(3 * NROWS, 128)

    bblk = eid >> 1
    boff = (eid & 1) * 64
    pltpu.sync_copy(bidx_hbm.at[pl.ds(bblk * 256 + boff, BP)], ba_v)
    pltpu.sync_copy(bidx_hbm.at[pl.ds(bblk * 256 + 128 + boff, BP)], bb_v)
    pltpu.sync_copy(bpar_hbm.at[pl.ds(bblk * 256 + boff, BP)], bk0_v)
    pltpu.sync_copy(bpar_hbm.at[pl.ds(bblk * 256 + 128 + boff, BP)], breq_v)
    pltpu.sync_copy(aidx_hbm.at[pl.ds(eid * 512, 128)], a1_v)
    pltpu.sync_copy(aidx_hbm.at[pl.ds(eid * 512 + 128, 128)], a2_v)
    pltpu.sync_copy(aidx_hbm.at[pl.ds(eid * 512 + 256, 128)], a3_v)
    pltpu.sync_copy(apar_hbm.at[pl.ds(eid * 256, 128)], ak0_v)
    pltpu.sync_copy(apar_hbm.at[pl.ds(eid * 256 + 128, 128)], ath0_v)

    for j in range(BG):
        s = pl.ds(j * 16, 16)
        a = ba_v[s]
        b = bb_v[s]
        t = _trow(a, b)
        bidx_v[0, s] = t
        bidx_v[1, s] = t
        bidx_v[2, s] = t + NROWS
        bidx_v[3, s] = t + 2 * NROWS
        blane_v[s] = b & 127
    for j in range(AG):
        s = pl.ds(j * 16, 16)
        i1 = a1_v[s]
        i2 = a2_v[s]
        i3 = a3_v[s]
        t21 = _trow(i2, i1)
        t23 = _trow(i2, i3)
        aidx_v[0, s] = t21
        aidx_v[1, s] = t23
        aidx_v[2, s] = t21
        aidx_v[3, s] = t23
        aidx_v[4, s] = t21 + NROWS
        aidx_v[5, s] = t23 + NROWS
        aidx_v[6, s] = t21 + 2 * NROWS
        aidx_v[7, s] = t23 + 2 * NROWS
        lane1_v[s] = i1 & 127
        lane3_v[s] = i3 & 127

    def _zero(i, _):
        acc_v[pl.ds(i * 16, 16)] = jnp.zeros((16,), jnp.float32)
        return _
    lax.fori_loop(0, ACC // 16, _zero, None)

    rbufs = [rb0_v, rb1_v, rb2_v, rb3_v]
    sems = [sem0, sem1, sem2, sem3]
    passes = [
        (dist_t, True, 0, dvalb_v, blane_v),
        (vec_t, True, 1, vbx_v, blane_v),
        (vec_t, True, 2, vby_v, blane_v),
        (vec_t, True, 3, vbz_v, blane_v),
        (dist_t, False, 0, dval21_v, lane1_v),
        (dist_t, False, 1, dval23_v, lane3_v),
        (vec_t, False, 2, v21x_v, lane1_v),
        (vec_t, False, 3, v23x_v, lane3_v),
        (vec_t, False, 4, v21y_v, lane1_v),
        (vec_t, False, 5, v23y_v, lane3_v),
        (vec_t, False, 6, v21z_v, lane1_v),
        (vec_t, False, 7, v23z_v, lane3_v),
    ]
    NP = len(passes)

    def _fire(i):
        tbl, is_bond, k, _, _ = passes[i]
        buf = rbufs[i % NBUF]
        s = sems[i % NBUF]
        if is_bond:
            return pltpu.async_copy(
                tbl.at[bidx_v.at[k]], buf.at[pl.ds(0, BP)], s)
        return pltpu.async_copy(tbl.at[aidx_v.at[k]], buf.at[pl.ds(0, AP)], s)

    iota = lax.iota(jnp.int32, 16)
    cps = {i: _fire(i) for i in range(min(NBUF - 1, NP))}
    for i in range(NP):
        cps[i].wait()
        _, is_bond, _, dst, laner = passes[i]
        buf = rbufs[i % NBUF]
        for g in range(BG if is_bond else AG):
            so = pl.ds(g * 16, 16)
            dst[so] = plsc.load_gather(buf, [iota + g * 16, laner[so]])
        if i + NBUF - 1 < NP:
            cps[i + NBUF - 1] = _fire(i + NBUF - 1)

    evec = jnp.zeros((16,), jnp.float32)

    for j in range(BG):
        s = pl.ds(j * 16, 16)
        d = dvalb_v[s]
        k0 = bk0_v[s]
        x = d - breq_v[s]
        evec = evec + k0 * x * x
        f = 2.0 * k0 * x
        fx = f * vbx_v[s]
        fy = f * vby_v[s]
        fz = f * vbz_v[s]
        a = ba_v[s]
        b = bb_v[s]
        ia = ((a >> 7) << 9) + (a & 127)
        ib = ((b >> 7) << 9) + (b & 127)
        plsc.addupdate_scatter(acc_v, [ia], fx)
        plsc.addupdate_scatter(acc_v, [ia + 128], fy)
        plsc.addupdate_scatter(acc_v, [ia + 256], fz)
        plsc.addupdate_scatter(acc_v, [ib], -fx)
        plsc.addupdate_scatter(acc_v, [ib + 128], -fy)
        plsc.addupdate_scatter(acc_v, [ib + 256], -fz)

    for j in range(AG):
        s = pl.ds(j * 16, 16)
        d21 = dval21_v[s]
        d23 = dval23_v[s]
        x21 = v21x_v[s]
        y21 = v21y_v[s]
        z21 = v21z_v[s]
        x23 = v23x_v[s]
        y23 = v23y_v[s]
        z23 = v23z_v[s]
        cos = x21 * x23 + y21 * y23 + z21 * z23
        cos = jnp.minimum(jnp.maximum(cos, -1.0), 1.0)
        theta = _acos(cos)
        k0 = ak0_v[s]
        dth = theta - ath0_v[s]
        evec = evec + k0 * dth * dth
        sin = _sqrt(1.0 - cos * cos)
        coef = (-2.0 * k0 * dth) / sin
        coef = jnp.where(coef != coef, jnp.float32(0.0), coef)
        coef = jnp.minimum(jnp.maximum(coef, -FMAX), FMAX)
        c21 = coef / d21
        c23 = coef / d23
        f0x = c21 * (cos * x21 - x23)
        f0y = c21 * (cos * y21 - y23)
        f0z = c21 * (cos * z21 - z23)
        f2x = c23 * (cos * x23 - x21)
        f2y = c23 * (cos * y23 - y21)
        f2z = c23 * (cos * z23 - z21)
        w1 = a1_v[s]
        w2 = a2_v[s]
        w3 = a3_v[s]
        i1 = ((w1 >> 7) << 9) + (w1 & 127)
        i2 = ((w2 >> 7) << 9) + (w2 & 127)
        i3 = ((w3 >> 7) << 9) + (w3 & 127)
        plsc.addupdate_scatter(acc_v, [i1], f0x)
        plsc.addupdate_scatter(acc_v, [i1 + 128], f0y)
        plsc.addupdate_scatter(acc_v, [i1 + 256], f0z)
        plsc.addupdate_scatter(acc_v, [i2], -(f0x + f2x))
        plsc.addupdate_scatter(acc_v, [i2 + 128], -(f0y + f2y))
        plsc.addupdate_scatter(acc_v, [i2 + 256], -(f0z + f2z))
        plsc.addupdate_scatter(acc_v, [i3], f2x)
        plsc.addupdate_scatter(acc_v, [i3 + 128], f2y)
        plsc.addupdate_scatter(acc_v, [i3 + 256], f2z)

    acc_v[pl.ds(E_SLOT, 16)] = evec

    for c in range(NS):
        pltpu.sync_copy(acc_v.at[pl.ds(c * CHUNK, CHUNK)],
                        spmem.at[c, sid])
    plsc.subcore_barrier()
    pltpu.sync_copy(spmem.at[sid], blk_v)

    def _sum(i, _):
        off = i * 16
        tot = blk_v[0, pl.ds(off, 16)]
        for t in range(1, NS):
            tot = tot + blk_v[t, pl.ds(off, 16)]
        blk_v[0, pl.ds(off, 16)] = tot
        return _
    lax.fori_loop(0, CHUNK // 16, _sum, None)

    @pl.when(sid == E_TILE)
    def _finish_energy():
        loc = pl.ds(E_SLOT - E_TILE * CHUNK, 16)
        ev = blk_v[0, loc]
        blk_v[0, loc] = jnp.broadcast_to(jnp.sum(ev), (16,))

    pltpu.sync_copy(blk_v.at[0], out_hbm.at[cid, pl.ds(sid * CHUNK, CHUNK)])


@jax.jit
def kernel(dist_mat, vector_mat, bond_params, angle_params, bond_idx, angle_idx):
    mesh = plsc.VectorSubcoreMesh(
        core_axis_name="c", subcore_axis_name="s", num_cores=2)
    sc_fn = pl.kernel(
        _sc_body,
        out_type=jax.ShapeDtypeStruct((NC, ACC), jnp.float32),
        mesh=mesh,
        compiler_params=pltpu.CompilerParams(needs_layout_passes=False),
        scratch_types=[
            pltpu.VMEM((BP,), jnp.int32),
            pltpu.VMEM((BP,), jnp.int32),
            pltpu.VMEM((BP,), jnp.float32),
            pltpu.VMEM((BP,), jnp.float32),
            pltpu.VMEM((AP,), jnp.int32),
            pltpu.VMEM((AP,), jnp.int32),
            pltpu.VMEM((AP,), jnp.int32),
            pltpu.VMEM((AP,), jnp.float32),
            pltpu.VMEM((AP,), jnp.float32),
            pltpu.VMEM((4, BP), jnp.int32),
            pltpu.VMEM((8, AP), jnp.int32),
            pltpu.VMEM((BP,), jnp.int32),
            pltpu.VMEM((AP,), jnp.int32),
            pltpu.VMEM((AP,), jnp.int32),
            pltpu.VMEM((128, 128), jnp.float32),
            pltpu.VMEM((128, 128), jnp.float32),
            pltpu.VMEM((128, 128), jnp.float32),
            pltpu.VMEM((128, 128), jnp.float32),
            pltpu.VMEM((BP,), jnp.float32),
            pltpu.VMEM((AP,), jnp.float32),
            pltpu.VMEM((AP,), jnp.float32),
            pltpu.VMEM((BP,), jnp.float32),
            pltpu.VMEM((BP,), jnp.float32),
            pltpu.VMEM((BP,), jnp.float32),
            pltpu.VMEM((AP,), jnp.float32),
            pltpu.VMEM((AP,), jnp.float32),
            pltpu.VMEM((AP,), jnp.float32),
            pltpu.VMEM((AP,), jnp.float32),
            pltpu.VMEM((AP,), jnp.float32),
            pltpu.VMEM((AP,), jnp.float32),
            pltpu.VMEM((ACC,), jnp.float32),
            pltpu.VMEM((NS, CHUNK), jnp.float32),
            pltpu.VMEM_SHARED((NS, NS, CHUNK), jnp.float32),
            pltpu.SemaphoreType.DMA,
            pltpu.SemaphoreType.DMA,
            pltpu.SemaphoreType.DMA,
            pltpu.SemaphoreType.DMA,
        ],
    )
    f32 = jnp.float32
    i32 = jnp.int32
    dist4 = dist_mat.reshape(N_ATOMS // 8, 8, N_ATOMS // 128, 128
                             ).transpose(0, 2, 1, 3)
    vec5 = vector_mat.transpose(2, 0, 1).reshape(
        3, N_ATOMS // 8, 8, N_ATOMS // 128, 128).transpose(0, 1, 3, 2, 4)
    bidx_f = bond_idx.astype(i32).reshape(16, 128, 2
                                          ).transpose(0, 2, 1).reshape(-1)
    bpar_f = bond_params.astype(f32).reshape(16, 128, 2
                                             ).transpose(0, 2, 1).reshape(-1)
    aidx_p = jnp.pad(angle_idx.astype(i32), ((0, 0), (0, 1)))
    aidx_f = aidx_p.reshape(32, 128, 4).transpose(0, 2, 1).reshape(-1)
    apar_f = angle_params.astype(f32).reshape(32, 128, 2
                                              ).transpose(0, 2, 1).reshape(-1)
    out2 = sc_fn(dist4, vec5, bidx_f, bpar_f, aidx_f, apar_f)
    out = out2[0] + out2[1]
    energy = out[E_SLOT]
    forces = out[:FSLOTS].reshape(16, 4, 128).transpose(0, 2, 1
                                                        ).reshape(N_ATOMS, 4)[:, :3]
    return energy, forces


# speedup vs baseline: 1.5519x; 1.1861x over previous
import jax
import jax.numpy as jnp
import numpy as np
from jax import lax
from jax.experimental import pallas as pl
from jax.experimental.pallas import tpu as pltpu
from jax.experimental.pallas import tpu_sc as plsc

N_ATOMS = 2048
N_BONDS = 2048
N_ANGLES = 4096

NC = 2
NS = 16
NW = NC * NS
BP = N_BONDS // NW
AP = N_ANGLES // NW
BG = BP // 16
AG = AP // 16

NROWS = (N_ATOMS * N_ATOMS) // 128

FSLOTS = (N_ATOMS // 128) * 4 * 128
ACC = 10240
CHUNK = ACC // NS
E_SLOT = FSLOTS
E_TILE = E_SLOT // CHUNK
NBUF = 4

FMAX = np.float32(3.4028235e38)
PI = np.float32(3.14159265358979)
ACOS_C = [1.5707963050, -0.2145988016, 0.0889789874, -0.0501743046,
          0.0308918810, -0.0170881256, 0.0066700901, -0.0012624911]


def _sqrt(y):
    i = plsc.bitcast(y, jnp.int32)
    i = jnp.int32(0x5F3759DF) - (i >> 1)
    r = plsc.bitcast(i, jnp.float32)
    for _ in range(3):
        r = r * (1.5 - 0.5 * y * r * r)
    return y * r


def _acos(c):
    xa = jnp.abs(c)
    p = jnp.float32(ACOS_C[7])
    for a in ACOS_C[6::-1]:
        p = p * xa + jnp.float32(a)
    t = _sqrt(1.0 - xa) * p
    return jnp.where(c < 0, PI - t, t)


def _trow(a, b):
    return ((a >> 3) << 7) + ((b >> 7) << 3) + (a & 7)


def _sc_body(dist4_hbm, vec5_hbm, bidx_hbm, bpar_hbm, aidx_hbm, apar_hbm,
             out_hbm,
             ba_v, bb_v, bk0_v, breq_v,
             a1_v, a2_v, a3_v, ak0_v, ath0_v,
             bidx_v, aidx_v, blane_v, lane1_v, lane3_v,
             rb0_v, rb1_v, rb2_v, rb3_v,
             dvalb_v, dval21_v, dval23_v,
             vbx_v, vby_v, vbz_v,
             v21x_v, v21y_v, v21z_v,
             v23x_v, v23y_v, v23z_v,
             acc_v, blk_v, spmem, sem0, sem1, sem2, sem3):
    cid = lax.axis_index("c")
    sid = lax.axis_index("s")
    eid = cid * NS + sid
    dist_t = dist4_hbm.reshape(NROWS, 128)
    vec_t = vec5_hbm.reshape(3 * NROWS, 128)

    bblk = eid >> 1
    boff = (eid & 1) * 64
    pcps = [
        pltpu.async_copy(bidx_hbm.at[pl.ds(bblk * 256 + boff, BP)],
                         ba_v, sem0),
        pltpu.async_copy(bidx_hbm.at[pl.ds(bblk * 256 + 128 + boff, BP)],
                         bb_v, sem0),
        pltpu.async_copy(bpar_hbm.at[pl.ds(bblk * 256 + boff, BP)],
                         bk0_v, sem0),
        pltpu.async_copy(bpar_hbm.at[pl.ds(bblk * 256 + 128 + boff, BP)],
                         breq_v, sem0),
        pltpu.async_copy(aidx_hbm.at[pl.ds(eid * 512, 128)], a1_v, sem0),
        pltpu.async_copy(aidx_hbm.at[pl.ds(eid * 512 + 128, 128)],
                         a2_v, sem0),
        pltpu.async_copy(aidx_hbm.at[pl.ds(eid * 512 + 256, 128)],
                         a3_v, sem0),
        pltpu.async_copy(apar_hbm.at[pl.ds(eid * 256, 128)], ak0_v, sem0),
        pltpu.async_copy(apar_hbm.at[pl.ds(eid * 256 + 128, 128)],
                         ath0_v, sem0),
    ]
    for c in pcps:
        c.wait()

    for j in range(BG):
        s = pl.ds(j * 16, 16)
        a = ba_v[s]
        b = bb_v[s]
        t = _trow(a, b)
        bidx_v[0, s] = t
        bidx_v[1, s] = t
        bidx_v[2, s] = t + NROWS
        bidx_v[3, s] = t + 2 * NROWS
        blane_v[s] = b & 127
    for j in range(AG):
        s = pl.ds(j * 16, 16)
        i1 = a1_v[s]
        i2 = a2_v[s]
        i3 = a3_v[s]
        t21 = _trow(i2, i1)
        t23 = _trow(i2, i3)
        aidx_v[0, s] = t21
        aidx_v[1, s] = t23
        aidx_v[2, s] = t21
        aidx_v[3, s] = t23
        aidx_v[4, s] = t21 + NROWS
        aidx_v[5, s] = t23 + NROWS
        aidx_v[6, s] = t21 + 2 * NROWS
        aidx_v[7, s] = t23 + 2 * NROWS
        lane1_v[s] = i1 & 127
        lane3_v[s] = i3 & 127

    rbufs = [rb0_v, rb1_v, rb2_v, rb3_v]
    sems = [sem0, sem1, sem2, sem3]
    passes = [
        (dist_t, True, 0, dvalb_v, blane_v),
        (vec_t, True, 1, vbx_v, blane_v),
        (vec_t, True, 2, vby_v, blane_v),
        (vec_t, True, 3, vbz_v, blane_v),
        (dist_t, False, 0, dval21_v, lane1_v),
        (dist_t, False, 1, dval23_v, lane3_v),
        (vec_t, False, 2, v21x_v, lane1_v),
        (vec_t, False, 3, v23x_v, lane3_v),
        (vec_t, False, 4, v21y_v, lane1_v),
        (vec_t, False, 5, v23y_v, lane3_v),
        (vec_t, False, 6, v21z_v, lane1_v),
        (vec_t, False, 7, v23z_v, lane3_v),
    ]
    NP = len(passes)

    def _fire(i):
        tbl, is_bond, k, _, _ = passes[i]
        buf = rbufs[i % NBUF]
        s = sems[i % NBUF]
        if is_bond:
            return pltpu.async_copy(
                tbl.at[bidx_v.at[k]], buf.at[pl.ds(0, BP)], s)
        return pltpu.async_copy(tbl.at[aidx_v.at[k]], buf.at[pl.ds(0, AP)], s)

    iota = lax.iota(jnp.int32, 16)
    cps = {i: _fire(i) for i in range(min(NBUF - 1, NP))}

    def _zero(i, _):
        for u in range(4):
            acc_v[pl.ds(i * 64 + u * 16, 16)] = jnp.zeros((16,), jnp.float32)
        return _
    lax.fori_loop(0, ACC // 64, _zero, None)

    for i in range(NP):
        cps[i].wait()
        _, is_bond, _, dst, laner = passes[i]
        buf = rbufs[i % NBUF]
        for g in range(BG if is_bond else AG):
            so = pl.ds(g * 16, 16)
            dst[so] = plsc.load_gather(buf, [iota + g * 16, laner[so]])
        if i + NBUF - 1 < NP:
            cps[i + NBUF - 1] = _fire(i + NBUF - 1)

    evec = jnp.zeros((16,), jnp.float32)

    for j in range(BG):
        s = pl.ds(j * 16, 16)
        d = dvalb_v[s]
        k0 = bk0_v[s]
        x = d - breq_v[s]
        evec = evec + k0 * x * x
        f = 2.0 * k0 * x
        fx = f * vbx_v[s]
        fy = f * vby_v[s]
        fz = f * vbz_v[s]
        a = ba_v[s]
        b = bb_v[s]
        ia = ((a >> 7) << 9) + (a & 127)
        ib = ((b >> 7) << 9) + (b & 127)
        plsc.addupdate_scatter(acc_v, [ia], fx)
        plsc.addupdate_scatter(acc_v, [ia + 128], fy)
        plsc.addupdate_scatter(acc_v, [ia + 256], fz)
        plsc.addupdate_scatter(acc_v, [ib], -fx)
        plsc.addupdate_scatter(acc_v, [ib + 128], -fy)
        plsc.addupdate_scatter(acc_v, [ib + 256], -fz)

    for j in range(AG):
        s = pl.ds(j * 16, 16)
        d21 = dval21_v[s]
        d23 = dval23_v[s]
        x21 = v21x_v[s]
        y21 = v21y_v[s]
        z21 = v21z_v[s]
        x23 = v23x_v[s]
        y23 = v23y_v[s]
        z23 = v23z_v[s]
        cos = x21 * x23 + y21 * y23 + z21 * z23
        cos = jnp.minimum(jnp.maximum(cos, -1.0), 1.0)
        theta = _acos(cos)
        k0 = ak0_v[s]
        dth = theta - ath0_v[s]
        evec = evec + k0 * dth * dth
        sin = _sqrt(1.0 - cos * cos)
        coef = (-2.0 * k0 * dth) / sin
        coef = jnp.where(coef != coef, jnp.float32(0.0), coef)
        coef = jnp.minimum(jnp.maximum(coef, -FMAX), FMAX)
        c21 = coef / d21
        c23 = coef / d23
        f0x = c21 * (cos * x21 - x23)
        f0y = c21 * (cos * y21 - y23)
        f0z = c21 * (cos * z21 - z23)
        f2x = c23 * (cos * x23 - x21)
        f2y = c23 * (cos * y23 - y21)
        f2z = c23 * (cos * z23 - z21)
        w1 = a1_v[s]
        w2 = a2_v[s]
        w3 = a3_v[s]
        i1 = ((w1 >> 7) << 9) + (w1 & 127)
        i2 = ((w2 >> 7) << 9) + (w2 & 127)
        i3 = ((w3 >> 7) << 9) + (w3 & 127)
        plsc.addupdate_scatter(acc_v, [i1], f0x)
        plsc.addupdate_scatter(acc_v, [i1 + 128], f0y)
        plsc.addupdate_scatter(acc_v, [i1 + 256], f0z)
        plsc.addupdate_scatter(acc_v, [i2], -(f0x + f2x))
        plsc.addupdate_scatter(acc_v, [i2 + 128], -(f0y + f2y))
        plsc.addupdate_scatter(acc_v, [i2 + 256], -(f0z + f2z))
        plsc.addupdate_scatter(acc_v, [i3], f2x)
        plsc.addupdate_scatter(acc_v, [i3 + 128], f2y)
        plsc.addupdate_scatter(acc_v, [i3 + 256], f2z)

    acc_v[pl.ds(E_SLOT, 16)] = evec

    scps = [pltpu.async_copy(acc_v.at[pl.ds(c * CHUNK, CHUNK)],
                             spmem.at[c, sid], sem0) for c in range(NS)]
    for c in scps:
        c.wait()
    plsc.subcore_barrier()
    pltpu.sync_copy(spmem.at[sid], blk_v)

    def _sum(i, _):
        off = i * 16
        tot = blk_v[0, pl.ds(off, 16)]
        for t in range(1, NS):
            tot = tot + blk_v[t, pl.ds(off, 16)]
        blk_v[0, pl.ds(off, 16)] = tot
        return _
    lax.fori_loop(0, CHUNK // 16, _sum, None)

    @pl.when(sid == E_TILE)
    def _finish_energy():
        loc = pl.ds(E_SLOT - E_TILE * CHUNK, 16)
        ev = blk_v[0, loc]
        blk_v[0, loc] = jnp.broadcast_to(jnp.sum(ev), (16,))

    pltpu.sync_copy(blk_v.at[0], out_hbm.at[cid, pl.ds(sid * CHUNK, CHUNK)])


@jax.jit
def kernel(dist_mat, vector_mat, bond_params, angle_params, bond_idx, angle_idx):
    mesh = plsc.VectorSubcoreMesh(
        core_axis_name="c", subcore_axis_name="s", num_cores=2)
    sc_fn = pl.kernel(
        _sc_body,
        out_type=jax.ShapeDtypeStruct((NC, ACC), jnp.float32),
        mesh=mesh,
        compiler_params=pltpu.CompilerParams(needs_layout_passes=False),
        scratch_types=[
            pltpu.VMEM((BP,), jnp.int32),
            pltpu.VMEM((BP,), jnp.int32),
            pltpu.VMEM((BP,), jnp.float32),
            pltpu.VMEM((BP,), jnp.float32),
            pltpu.VMEM((AP,), jnp.int32),
            pltpu.VMEM((AP,), jnp.int32),
            pltpu.VMEM((AP,), jnp.int32),
            pltpu.VMEM((AP,), jnp.float32),
            pltpu.VMEM((AP,), jnp.float32),
            pltpu.VMEM((4, BP), jnp.int32),
            pltpu.VMEM((8, AP), jnp.int32),
            pltpu.VMEM((BP,), jnp.int32),
            pltpu.VMEM((AP,), jnp.int32),
            pltpu.VMEM((AP,), jnp.int32),
            pltpu.VMEM((128, 128), jnp.float32),
            pltpu.VMEM((128, 128), jnp.float32),
            pltpu.VMEM((128, 128), jnp.float32),
            pltpu.VMEM((128, 128), jnp.float32),
            pltpu.VMEM((BP,), jnp.float32),
            pltpu.VMEM((AP,), jnp.float32),
            pltpu.VMEM((AP,), jnp.float32),
            pltpu.VMEM((BP,), jnp.float32),
            pltpu.VMEM((BP,), jnp.float32),
            pltpu.VMEM((BP,), jnp.float32),
            pltpu.VMEM((AP,), jnp.float32),
            pltpu.VMEM((AP,), jnp.float32),
            pltpu.VMEM((AP,), jnp.float32),
            pltpu.VMEM((AP,), jnp.float32),
            pltpu.VMEM((AP,), jnp.float32),
            pltpu.VMEM((AP,), jnp.float32),
            pltpu.VMEM((ACC,), jnp.float32),
            pltpu.VMEM((NS, CHUNK), jnp.float32),
            pltpu.VMEM_SHARED((NS, NS, CHUNK), jnp.float32),
            pltpu.SemaphoreType.DMA,
            pltpu.SemaphoreType.DMA,
            pltpu.SemaphoreType.DMA,
            pltpu.SemaphoreType.DMA,
        ],
    )
    f32 = jnp.float32
    i32 = jnp.int32
    dist4 = dist_mat.reshape(N_ATOMS // 8, 8, N_ATOMS // 128, 128
                             ).transpose(0, 2, 1, 3)
    vec5 = vector_mat.transpose(2, 0, 1).reshape(
        3, N_ATOMS // 8, 8, N_ATOMS // 128, 128).transpose(0, 1, 3, 2, 4)
    bidx_f = bond_idx.astype(i32).reshape(16, 128, 2
                                          ).transpose(0, 2, 1).reshape(-1)
    bpar_f = bond_params.astype(f32).reshape(16, 128, 2
                                             ).transpose(0, 2, 1).reshape(-1)
    aidx_p = jnp.pad(angle_idx.astype(i32), ((0, 0), (0, 1)))
    aidx_f = aidx_p.reshape(32, 128, 4).transpose(0, 2, 1).reshape(-1)
    apar_f = angle_params.astype(f32).reshape(32, 128, 2
                                              ).transpose(0, 2, 1).reshape(-1)
    out2 = sc_fn(dist4, vec5, bidx_f, bpar_f, aidx_f, apar_f)
    out = out2[0] + out2[1]
    energy = out[E_SLOT]
    forces = out[:FSLOTS].reshape(16, 4, 128).transpose(0, 2, 1
                                                        ).reshape(N_ATOMS, 4)[:, :3]
    return energy, forces
